# baseline (speedup 1.0000x reference)
import functools

import jax
import jax.numpy as jnp
from jax.experimental import pallas as pl
from jax.experimental.pallas import tpu as pltpu


def _make_body(S: int, M: int, nb: int):
    R = 2 * M + 1

    def body(x_ref, rk_ref, o_ref, bias_ref):
        @pl.when(pl.program_id(1) == 0)
        def _():
            i = jax.lax.broadcasted_iota(jnp.int32, (S, R), 0)
            r = jax.lax.broadcasted_iota(jnp.int32, (S, R), 1)
            j = i - (r - M)
            c = ((j >= 0) & (j < S)).astype(jnp.float32)
            left = jnp.maximum(0, S - i - M).astype(jnp.float32)
            right = jnp.maximum(0, i - M + 1).astype(jnp.float32)
            c = jnp.where(r == 0, left, jnp.where(r == R - 1, right, c))
            b = jnp.dot(c, rk_ref[...], preferred_element_type=jnp.float32)
            for k in range(nb):
                bias_ref[k * S:(k + 1) * S, :] = b

        o_ref[...] = x_ref[...] + bias_ref[...]

    return body


@functools.partial(jax.jit, static_argnames=("max_rel_dist", "batch_per_block"))
def _pe_relative(x, rel_k, *, max_rel_dist, batch_per_block=1):
    B, S, D = x.shape
    M = max_rel_dist
    R = 2 * M + 1

    x2d = x.reshape(B * S, D)

    nb = batch_per_block
    while B % (2 * nb) != 0:
        nb -= 1
    rows = nb * S
    per_core = B // (2 * nb)

    out = pl.pallas_call(
        _make_body(S, M, nb),
        out_shape=jax.ShapeDtypeStruct((B * S, D), x.dtype),
        grid_spec=pltpu.PrefetchScalarGridSpec(
            num_scalar_prefetch=0,
            grid=(2, per_core),
            in_specs=[
                pl.BlockSpec((rows, D), lambda c, b, n=per_core: (c * n + b, 0)),
                pl.BlockSpec((R, D), lambda c, b: (0, 0)),
            ],
            out_specs=pl.BlockSpec((rows, D), lambda c, b, n=per_core: (c * n + b, 0)),
            scratch_shapes=[pltpu.VMEM((rows, D), jnp.float32)],
        ),
        compiler_params=pltpu.CompilerParams(
            dimension_semantics=("parallel", "arbitrary")),
    )(x2d, rel_k.astype(jnp.float32))
    return out.reshape(B, S, D)


def kernel(x, rel_k):
    return _pe_relative(x, rel_k, max_rel_dist=128)


# speedup vs baseline: 1.1552x; 1.1552x over previous
import functools

import jax
import jax.numpy as jnp
from jax.experimental import pallas as pl
from jax.experimental.pallas import tpu as pltpu


def _make_body(S: int, M: int, nb: int):
    R = 2 * M + 1

    def body(x_ref, rk_ref, o_ref, bias_ref):
        @pl.when(pl.program_id(1) == 0)
        def _():
            i = jax.lax.broadcasted_iota(jnp.int32, (S, R), 0)
            r = jax.lax.broadcasted_iota(jnp.int32, (S, R), 1)
            j = i - (r - M)
            c = ((j >= 0) & (j < S)).astype(jnp.float32)
            left = jnp.maximum(0, S - i - M).astype(jnp.float32)
            right = jnp.maximum(0, i - M + 1).astype(jnp.float32)
            c = jnp.where(r == 0, left, jnp.where(r == R - 1, right, c))
            b = jnp.dot(c, rk_ref[...], preferred_element_type=jnp.float32)
            for k in range(nb):
                bias_ref[k * S:(k + 1) * S, :] = b

        o_ref[...] = x_ref[...] + bias_ref[...]

    return body


@functools.partial(jax.jit, static_argnames=("max_rel_dist", "batch_per_block"))
def _pe_relative(x, rel_k, *, max_rel_dist, batch_per_block=4):
    B, S, D = x.shape
    M = max_rel_dist
    R = 2 * M + 1

    x2d = x.reshape(B * S, D)

    nb = batch_per_block
    while B % (2 * nb) != 0:
        nb -= 1
    rows = nb * S
    per_core = B // (2 * nb)

    out = pl.pallas_call(
        _make_body(S, M, nb),
        out_shape=jax.ShapeDtypeStruct((B * S, D), x.dtype),
        grid_spec=pltpu.PrefetchScalarGridSpec(
            num_scalar_prefetch=0,
            grid=(2, per_core),
            in_specs=[
                pl.BlockSpec((rows, D), lambda c, b, n=per_core: (c * n + b, 0)),
                pl.BlockSpec((R, D), lambda c, b: (0, 0)),
            ],
            out_specs=pl.BlockSpec((rows, D), lambda c, b, n=per_core: (c * n + b, 0)),
            scratch_shapes=[pltpu.VMEM((rows, D), jnp.float32)],
        ),
        compiler_params=pltpu.CompilerParams(
            dimension_semantics=("parallel", "arbitrary")),
    )(x2d, rel_k.astype(jnp.float32))
    return out.reshape(B, S, D)


def kernel(x, rel_k):
    return _pe_relative(x, rel_k, max_rel_dist=128)
